# Initial kernel scaffold; baseline (speedup 1.0000x reference)
import jax
import jax.numpy as jnp
from jax.experimental import pallas as pl


def kernel(img_feat_map, bev_feat_map, top_anchors, image_shape, calibration_dict, ground_plane, img_mask, bev_mask, W_cls, b_cls, W_off, b_off, W_ang, b_ang):
    raise NotImplementedError("write your pallas kernel here")


# speedup vs baseline: 1.0102x; 1.0102x over previous
import functools

import jax
import jax.numpy as jnp
from jax import lax
from jax.experimental import pallas as pl
from jax.experimental.pallas import tpu as pltpu

_EXT_X0, _EXT_X1 = -40.0, 40.0
_EXT_Z0, _EXT_Z1 = 0.0, 70.0
_ROI = 7
_C = 32
_NMS_THR = 0.01
_NMS_OUT = 100
_NP = 20480
_ROWS = _NP // 128

_SX = (-1.0, 1.0, 1.0, -1.0)
_SZ = (-1.0, -1.0, 1.0, 1.0)


def _mlp_body(x_ref, w_ref, b_ref, o_ref):
    o_ref[...] = (
        jnp.dot(x_ref[...].astype(jnp.bfloat16),
                w_ref[...].astype(jnp.bfloat16),
                preferred_element_type=jnp.float32)
        + b_ref[...]
    )


def _mlp(fused, w, b, block=400):
    n, k = fused.shape
    m = w.shape[1]
    grid = n // block
    return pl.pallas_call(
        _mlp_body,
        grid=(grid,),
        in_specs=[
            pl.BlockSpec((block, k), lambda i: (i, 0)),
            pl.BlockSpec((k, m), lambda i: (0, 0)),
            pl.BlockSpec((1, m), lambda i: (0, 0)),
        ],
        out_specs=pl.BlockSpec((block, m), lambda i: (i, 0)),
        out_shape=jax.ShapeDtypeStruct((n, m), jnp.float32),
    )(fused, w, b.reshape(1, m))


def _nms_body(anc_ref, off_ref, sc_ref, out_ref):
    x = anc_ref[0]
    z = anc_ref[2]
    dx = anc_ref[3]
    dz = anc_ref[5]

    pxs = [x + 0.5 * _SX[k] * dx + off_ref[4 + k] for k in range(4)]
    pzs = [z + 0.5 * _SZ[k] * dz + off_ref[8 + k] for k in range(4)]
    xm = (pxs[0] + pxs[1] + pxs[2] + pxs[3]) * 0.25
    zm = (pzs[0] + pzs[1] + pzs[2] + pzs[3]) * 0.25
    dxn = jnp.maximum(jnp.maximum(pxs[0], pxs[1]), jnp.maximum(pxs[2], pxs[3])) - \
        jnp.minimum(jnp.minimum(pxs[0], pxs[1]), jnp.minimum(pxs[2], pxs[3]))
    dzn = jnp.maximum(jnp.maximum(pzs[0], pzs[1]), jnp.maximum(pzs[2], pzs[3])) - \
        jnp.minimum(jnp.minimum(pzs[0], pzs[1]), jnp.minimum(pzs[2], pzs[3]))
    bx1 = xm - dxn * 0.5
    bx2 = xm + dxn * 0.5
    bz1 = zm - dzn * 0.5
    bz2 = zm + dzn * 0.5
    areas = (bx2 - bx1) * (bz2 - bz1)

    scores = sc_ref[...]
    flatpos = (lax.broadcasted_iota(jnp.int32, (_ROWS, 128), 0) * 128
               + lax.broadcasted_iota(jnp.int32, (_ROWS, 128), 1))
    accpos = (lax.broadcasted_iota(jnp.int32, (8, 128), 0) * 128
              + lax.broadcasted_iota(jnp.int32, (8, 128), 1))

    def body(k, carry):
        valid, acc = carry
        s = jnp.where(valid > 0.5, scores, -1e30)
        m = jnp.max(s)
        idx = jnp.min(jnp.where(s == m, flatpos, jnp.int32(2147483647)))
        sel = flatpos == idx
        x1i = jnp.sum(jnp.where(sel, bx1, 0.0))
        z1i = jnp.sum(jnp.where(sel, bz1, 0.0))
        x2i = jnp.sum(jnp.where(sel, bx2, 0.0))
        z2i = jnp.sum(jnp.where(sel, bz2, 0.0))
        ai = jnp.sum(jnp.where(sel, areas, 0.0))
        xx1 = jnp.maximum(x1i, bx1)
        zz1 = jnp.maximum(z1i, bz1)
        xx2 = jnp.minimum(x2i, bx2)
        zz2 = jnp.minimum(z2i, bz2)
        inter = jnp.maximum(xx2 - xx1, 0.0) * jnp.maximum(zz2 - zz1, 0.0)
        iou = inter / (ai + areas - inter + 1e-8)
        keep = (iou <= _NMS_THR) & (~sel)
        valid = jnp.where(keep, valid, 0.0)
        acc = jnp.where(accpos == k, idx, acc)
        return valid, acc

    valid0 = jnp.ones((_ROWS, 128), dtype=jnp.float32)
    acc0 = jnp.zeros((8, 128), dtype=jnp.int32)
    _, acc = lax.fori_loop(0, _NMS_OUT, body, (valid0, acc0))
    out_ref[...] = acc


def _nms(anc, off12, scores):
    return pl.pallas_call(
        _nms_body,
        in_specs=[
            pl.BlockSpec((6, _ROWS, 128), lambda: (0, 0, 0)),
            pl.BlockSpec((12, _ROWS, 128), lambda: (0, 0, 0)),
            pl.BlockSpec((_ROWS, 128), lambda: (0, 0)),
        ],
        out_specs=pl.BlockSpec((8, 128), lambda: (0, 0)),
        out_shape=jax.ShapeDtypeStruct((8, 128), jnp.int32),
    )(anc, off12, scores)


def _bev_norm_boxes(anchors):
    x, z = anchors[:, 0], anchors[:, 2]
    dx, dz = anchors[:, 3], anchors[:, 5]
    u1 = (x - dx / 2 - _EXT_X0) / (_EXT_X1 - _EXT_X0)
    u2 = (x + dx / 2 - _EXT_X0) / (_EXT_X1 - _EXT_X0)
    v1 = (_EXT_Z1 - (z + dz / 2)) / (_EXT_Z1 - _EXT_Z0)
    v2 = (_EXT_Z1 - (z - dz / 2)) / (_EXT_Z1 - _EXT_Z0)
    return jnp.clip(jnp.stack([u1, v1, u2, v2], axis=1), 0.0, 1.0)


def _img_norm_boxes(anchors, image_shape, P):
    x, y, z = anchors[:, 0], anchors[:, 1], anchors[:, 2]
    dx, dy, dz = anchors[:, 3], anchors[:, 4], anchors[:, 5]
    offs = jnp.array([[sx, sy, sz] for sx in (-0.5, 0.5) for sy in (-0.5, 0.5)
                      for sz in (-0.5, 0.5)], dtype=jnp.float32)
    cx = x[:, None] + offs[None, :, 0] * dx[:, None]
    cy = y[:, None] + offs[None, :, 1] * dy[:, None]
    cz = z[:, None] + offs[None, :, 2] * dz[:, None]
    pts = jnp.stack([cx, cy, cz, jnp.ones_like(cx)], axis=-1)
    proj = jnp.einsum('nkj,ij->nki', pts, P)
    u = proj[..., 0] / jnp.maximum(proj[..., 2], 1e-3)
    v = proj[..., 1] / jnp.maximum(proj[..., 2], 1e-3)
    b = jnp.stack([u.min(1), v.min(1), u.max(1), v.max(1)], axis=1)
    H, W = image_shape[0], image_shape[1]
    b = b / jnp.stack([W, H, W, H])
    return jnp.clip(b, 0.0, 1.0)


def _crop_resize(img, boxes):
    im = img[0]
    H, W = im.shape[0], im.shape[1]
    y1, x1, y2, x2 = boxes[:, 0], boxes[:, 1], boxes[:, 2], boxes[:, 3]
    t = jnp.linspace(0.0, 1.0, _ROI)
    ys = (y1[:, None] + t[None, :] * (y2 - y1)[:, None]) * (H - 1)
    xs = (x1[:, None] + t[None, :] * (x2 - x1)[:, None]) * (W - 1)
    y0 = jnp.clip(jnp.floor(ys), 0, H - 2)
    x0 = jnp.clip(jnp.floor(xs), 0, W - 2)
    wy = jnp.clip(ys - y0, 0.0, 1.0)[:, :, None, None]
    wx = jnp.clip(xs - x0, 0.0, 1.0)[:, None, :, None]
    y0i = y0.astype(jnp.int32)
    x0i = x0.astype(jnp.int32)

    def g(yy, xx):
        return im[yy[:, :, None], xx[:, None, :], :]

    v00 = g(y0i, x0i)
    v01 = g(y0i, x0i + 1)
    v10 = g(y0i + 1, x0i)
    v11 = g(y0i + 1, x0i + 1)
    return (v00 * (1 - wy) * (1 - wx) + v01 * (1 - wy) * wx
            + v10 * wy * (1 - wx) + v11 * wy * wx)


def _ground_y(gp, x, z):
    return -(gp[0] * x + gp[2] * z + gp[3]) / gp[1]


def _pad_t(a, fill=0.0):
    n, k = a.shape
    a = jnp.pad(a.T, ((0, 0), (0, _NP - n)), constant_values=fill)
    return a.reshape(k, _ROWS, 128)


def kernel(img_feat_map, bev_feat_map, top_anchors, image_shape,
           calibration_dict, ground_plane, img_mask, bev_mask,
           W_cls, b_cls, W_off, b_off, W_ang, b_ang):
    n = top_anchors.shape[0]
    gp = ground_plane

    bev_ins = _bev_norm_boxes(top_anchors)
    rgb_ins = _img_norm_boxes(top_anchors, image_shape, calibration_dict[0])
    boxes_img = jnp.take(rgb_ins, jnp.array([1, 0, 3, 2]), axis=1)
    boxes_bev = jnp.take(bev_ins, jnp.array([1, 0, 3, 2]), axis=1)

    rois_rgb = _crop_resize(img_mask * img_feat_map, boxes_img)
    rois_bev = _crop_resize(bev_mask * bev_feat_map, boxes_bev)
    fused = (rois_rgb + rois_bev) / (img_mask + bev_mask)
    fused = fused.reshape(n, _ROI * _ROI * _C)

    w_all = jnp.concatenate([W_cls, W_off, W_ang], axis=1)
    b_all = jnp.concatenate([b_cls, b_off, b_ang], axis=0)
    scores16 = _mlp(fused, w_all, b_all)
    obj = scores16[:, 0:4]
    off = scores16[:, 4:14]
    ang = scores16[:, 14:16]

    nms_score = jnp.max(obj[:, 1:], axis=1)

    anc_p = _pad_t(top_anchors)
    off_p = _pad_t(off)
    off12 = jnp.concatenate(
        [jnp.zeros((4, _ROWS, 128), jnp.float32), off_p[:8]], axis=0)
    sc_p = jnp.pad(nms_score, (0, _NP - n),
                   constant_values=-1e30).reshape(_ROWS, 128)

    acc = _nms(anc_p, off12, sc_p)
    top_idx = acc.reshape(-1)[:_NMS_OUT]

    obj_i = jnp.take(obj, top_idx, axis=0)
    top_scores_soft = jax.nn.softmax(obj_i, axis=1)
    ang_i = jnp.take(ang, top_idx, axis=0)
    top_orient = jnp.arctan2(ang_i[:, 1], ang_i[:, 0])

    a_i = jnp.take(top_anchors, top_idx, axis=0)
    o_i = jnp.take(off, top_idx, axis=0)
    x, y, z = a_i[:, 0], a_i[:, 1], a_i[:, 2]
    dx, dy, dz = a_i[:, 3], a_i[:, 4], a_i[:, 5]
    xs = jnp.stack([x - dx / 2, x + dx / 2, x + dx / 2, x - dx / 2], axis=1)
    zs = jnp.stack([z - dz / 2, z - dz / 2, z + dz / 2, z + dz / 2], axis=1)
    yg = _ground_y(gp, x, z)
    h1 = (y - dy / 2) - yg
    h2 = (y + dy / 2) - yg
    prop4cp = jnp.concatenate([xs, zs, h1[:, None], h2[:, None]], axis=1)
    pred4c = prop4cp + o_i
    pxs, pzs = pred4c[:, :4], pred4c[:, 4:8]
    ph1, ph2 = pred4c[:, 8], pred4c[:, 9]
    px, pz = pxs.mean(1), pzs.mean(1)
    pdx = pxs.max(1) - pxs.min(1)
    pdz = pzs.max(1) - pzs.min(1)
    pyg = _ground_y(gp, px, pz)
    py = pyg + (ph1 + ph2) / 2
    pdy = ph2 - ph1
    pred_anchors = jnp.stack([px, py, pz, pdx, pdy, pdz], axis=1)
    pred_box = jnp.concatenate(
        [pred_anchors, jnp.zeros((pred_anchors.shape[0], 1))], axis=1)

    return (top_scores_soft, (pred_anchors, pred4c, pred_box),
            top_orient, None)


# speedup vs baseline: 13.0894x; 12.9576x over previous
import functools

import jax
import jax.numpy as jnp
from jax import lax
from jax.experimental import pallas as pl
from jax.experimental.pallas import tpu as pltpu
from jax.experimental.pallas import tpu_sc as plsc

_EXT_X0, _EXT_X1 = -40.0, 40.0
_EXT_Z0, _EXT_Z1 = 0.0, 70.0
_ROI = 7
_PIX = _ROI * _ROI
_C = 32
_NMS_THR = 0.01
_NMS_OUT = 100
_N = 20000
_NP = 20480
_ROWS = _NP // 128
_NW = 32
_NT = _NP // _NW
_CH = 128
_NCHUNK = _NT // _CH

_SX = (-1.0, 1.0, 1.0, -1.0)
_SZ = (-1.0, -1.0, 1.0, 1.0)


def _roi_body(timg, tbev, ii, ib, wi, wb, f_out,
              rimg_v, rbev_v, ivi_v, ivb_v, wi_v, wb_v, out_v, sem):
    wid = lax.axis_index("s") * 2 + lax.axis_index("c")
    tbase = wid * _NT
    iota = lax.iota(jnp.int32, 16)

    def p_loop(p, _):
        pltpu.sync_copy(wi.at[:, p, pl.ds(tbase, _NT)], wi_v)
        pltpu.sync_copy(wb.at[:, p, pl.ds(tbase, _NT)], wb_v)

        def c_loop(c, _):
            pltpu.sync_copy(ii.at[p, pl.ds(tbase + c * _CH, _CH)], ivi_v)
            pltpu.sync_copy(ib.at[p, pl.ds(tbase + c * _CH, _CH)], ivb_v)
            cp1 = pltpu.async_copy(timg.at[ivi_v], rimg_v, sem)
            cp2 = pltpu.async_copy(tbev.at[ivb_v], rbev_v, sem)
            cp1.wait()
            cp2.wait()

            def g_loop(g, _):
                base = c * _CH + g * 16
                pix = g * 16 + iota
                ayi = wi_v[0, pl.ds(base, 16)]
                byi = wi_v[1, pl.ds(base, 16)]
                axi = wi_v[2, pl.ds(base, 16)]
                bxi = wi_v[3, pl.ds(base, 16)]
                ayb = wb_v[0, pl.ds(base, 16)]
                byb = wb_v[1, pl.ds(base, 16)]
                axb = wb_v[2, pl.ds(base, 16)]
                bxb = wb_v[3, pl.ds(base, 16)]
                for ch in range(_C):
                    gi = [plsc.load_gather(
                        rimg_v, [pix, jnp.full((16,), k * _C + ch, jnp.int32)])
                        for k in range(4)]
                    gb = [plsc.load_gather(
                        rbev_v, [pix, jnp.full((16,), k * _C + ch, jnp.int32)])
                        for k in range(4)]
                    a = (((gi[0] * ayi) * axi + (gi[1] * ayi) * bxi)
                         + (gi[2] * byi) * axi) + (gi[3] * byi) * bxi
                    b = (((gb[0] * ayb) * axb + (gb[1] * ayb) * bxb)
                         + (gb[2] * byb) * axb) + (gb[3] * byb) * bxb
                    out_v[ch, pl.ds(base, 16)] = a + b
                return 0

            lax.fori_loop(0, _CH // 16, g_loop, 0)
            return 0

        lax.fori_loop(0, _NCHUNK, c_loop, 0)
        pltpu.sync_copy(out_v, f_out.at[:, p, pl.ds(tbase, _NT)])
        return 0

    lax.fori_loop(0, _PIX, p_loop, 0)


def _roi_fuse(timg, tbev, ii, ib, wi, wb):
    mesh = plsc.VectorSubcoreMesh(core_axis_name="c", subcore_axis_name="s")
    k = functools.partial(
        pl.kernel,
        out_type=jax.ShapeDtypeStruct((_C, _PIX, _NP), jnp.float32),
        mesh=mesh,
        compiler_params=pltpu.CompilerParams(needs_layout_passes=False),
        scratch_types=[
            pltpu.VMEM((_CH, 128), jnp.float32),
            pltpu.VMEM((_CH, 128), jnp.float32),
            pltpu.VMEM((_CH,), jnp.int32),
            pltpu.VMEM((_CH,), jnp.int32),
            pltpu.VMEM((4, _NT), jnp.float32),
            pltpu.VMEM((4, _NT), jnp.float32),
            pltpu.VMEM((_C, _NT), jnp.float32),
            pltpu.SemaphoreType.DMA,
        ],
    )(_roi_body)
    return k(timg, tbev, ii, ib, wi, wb)


def _mlp_body(den_ref, wt_ref, b_ref, x_ref, o_ref):
    xd = (x_ref[...] / den_ref[0]).astype(jnp.bfloat16)
    o_ref[...] = (
        jnp.dot(wt_ref[...].astype(jnp.bfloat16), xd,
                preferred_element_type=jnp.float32)
        + b_ref[...]
    )


def _mlp_t(f2d, wt, b, den, block=512):
    grid = _NP // block
    return pl.pallas_call(
        _mlp_body,
        grid=(grid,),
        in_specs=[
            pl.BlockSpec(memory_space=pltpu.SMEM),
            pl.BlockSpec((16, f2d.shape[0]), lambda i: (0, 0)),
            pl.BlockSpec((16, 1), lambda i: (0, 0)),
            pl.BlockSpec((f2d.shape[0], block), lambda i: (0, i)),
        ],
        out_specs=pl.BlockSpec((16, block), lambda i: (0, i)),
        out_shape=jax.ShapeDtypeStruct((16, _NP), jnp.float32),
    )(den, wt, b.reshape(16, 1), f2d)


def _nms_body(anc_ref, st_ref, out_ref):
    x = anc_ref[0]
    z = anc_ref[2]
    dx = anc_ref[3]
    dz = anc_ref[5]

    pxs = [x + 0.5 * _SX[k] * dx + st_ref[4 + k] for k in range(4)]
    pzs = [z + 0.5 * _SZ[k] * dz + st_ref[8 + k] for k in range(4)]
    xm = (pxs[0] + pxs[1] + pxs[2] + pxs[3]) * 0.25
    zm = (pzs[0] + pzs[1] + pzs[2] + pzs[3]) * 0.25
    dxn = jnp.maximum(jnp.maximum(pxs[0], pxs[1]), jnp.maximum(pxs[2], pxs[3])) - \
        jnp.minimum(jnp.minimum(pxs[0], pxs[1]), jnp.minimum(pxs[2], pxs[3]))
    dzn = jnp.maximum(jnp.maximum(pzs[0], pzs[1]), jnp.maximum(pzs[2], pzs[3])) - \
        jnp.minimum(jnp.minimum(pzs[0], pzs[1]), jnp.minimum(pzs[2], pzs[3]))
    bx1 = xm - dxn * 0.5
    bx2 = xm + dxn * 0.5
    bz1 = zm - dzn * 0.5
    bz2 = zm + dzn * 0.5
    areas = (bx2 - bx1) * (bz2 - bz1)

    scores = jnp.maximum(jnp.maximum(st_ref[1], st_ref[2]), st_ref[3])
    flatpos = (lax.broadcasted_iota(jnp.int32, (_ROWS, 128), 0) * 128
               + lax.broadcasted_iota(jnp.int32, (_ROWS, 128), 1))
    accpos = (lax.broadcasted_iota(jnp.int32, (8, 128), 0) * 128
              + lax.broadcasted_iota(jnp.int32, (8, 128), 1))

    def body(k, carry):
        valid, acc = carry
        s = jnp.where(valid > 0.5, scores, -1e30)
        m = jnp.max(s)
        idx = jnp.min(jnp.where(s == m, flatpos, jnp.int32(2147483647)))
        sel = flatpos == idx
        x1i = jnp.sum(jnp.where(sel, bx1, 0.0))
        z1i = jnp.sum(jnp.where(sel, bz1, 0.0))
        x2i = jnp.sum(jnp.where(sel, bx2, 0.0))
        z2i = jnp.sum(jnp.where(sel, bz2, 0.0))
        ai = jnp.sum(jnp.where(sel, areas, 0.0))
        xx1 = jnp.maximum(x1i, bx1)
        zz1 = jnp.maximum(z1i, bz1)
        xx2 = jnp.minimum(x2i, bx2)
        zz2 = jnp.minimum(z2i, bz2)
        inter = jnp.maximum(xx2 - xx1, 0.0) * jnp.maximum(zz2 - zz1, 0.0)
        iou = inter / (ai + areas - inter + 1e-8)
        keep = (iou <= _NMS_THR) & (~sel)
        valid = jnp.where(keep, valid, 0.0)
        acc = jnp.where(accpos == k, idx, acc)
        return valid, acc

    valid0 = (flatpos < _N).astype(jnp.float32)
    acc0 = jnp.zeros((8, 128), dtype=jnp.int32)
    _, acc = lax.fori_loop(0, _NMS_OUT, body, (valid0, acc0))
    out_ref[...] = acc


def _nms(anc, st):
    return pl.pallas_call(
        _nms_body,
        in_specs=[
            pl.BlockSpec((6, _ROWS, 128), lambda: (0, 0, 0)),
            pl.BlockSpec((16, _ROWS, 128), lambda: (0, 0, 0)),
        ],
        out_specs=pl.BlockSpec((8, 128), lambda: (0, 0)),
        out_shape=jax.ShapeDtypeStruct((8, 128), jnp.int32),
    )(anc, st)


def _bev_norm_boxes(anchors):
    x, z = anchors[:, 0], anchors[:, 2]
    dx, dz = anchors[:, 3], anchors[:, 5]
    u1 = (x - dx / 2 - _EXT_X0) / (_EXT_X1 - _EXT_X0)
    u2 = (x + dx / 2 - _EXT_X0) / (_EXT_X1 - _EXT_X0)
    v1 = (_EXT_Z1 - (z + dz / 2)) / (_EXT_Z1 - _EXT_Z0)
    v2 = (_EXT_Z1 - (z - dz / 2)) / (_EXT_Z1 - _EXT_Z0)
    return jnp.clip(jnp.stack([u1, v1, u2, v2], axis=1), 0.0, 1.0)


def _img_norm_boxes(anchors, image_shape, P):
    x, y, z = anchors[:, 0], anchors[:, 1], anchors[:, 2]
    dx, dy, dz = anchors[:, 3], anchors[:, 4], anchors[:, 5]
    offs = jnp.array([[sx, sy, sz] for sx in (-0.5, 0.5) for sy in (-0.5, 0.5)
                      for sz in (-0.5, 0.5)], dtype=jnp.float32)
    cx = x[:, None] + offs[None, :, 0] * dx[:, None]
    cy = y[:, None] + offs[None, :, 1] * dy[:, None]
    cz = z[:, None] + offs[None, :, 2] * dz[:, None]
    pts = jnp.stack([cx, cy, cz, jnp.ones_like(cx)], axis=-1)
    proj = jnp.einsum('nkj,ij->nki', pts, P)
    u = proj[..., 0] / jnp.maximum(proj[..., 2], 1e-3)
    v = proj[..., 1] / jnp.maximum(proj[..., 2], 1e-3)
    b = jnp.stack([u.min(1), v.min(1), u.max(1), v.max(1)], axis=1)
    H, W = image_shape[0], image_shape[1]
    b = b / jnp.stack([W, H, W, H])
    return jnp.clip(b, 0.0, 1.0)


def _sample_coeffs(boxes, H, W):
    y1, x1, y2, x2 = boxes[:, 0], boxes[:, 1], boxes[:, 2], boxes[:, 3]
    t = jnp.linspace(0.0, 1.0, _ROI)
    ys = (y1[:, None] + t[None, :] * (y2 - y1)[:, None]) * (H - 1)
    xs = (x1[:, None] + t[None, :] * (x2 - x1)[:, None]) * (W - 1)
    y0 = jnp.clip(jnp.floor(ys), 0, H - 2)
    x0 = jnp.clip(jnp.floor(xs), 0, W - 2)
    wy = jnp.clip(ys - y0, 0.0, 1.0)
    wx = jnp.clip(xs - x0, 0.0, 1.0)
    y0i = y0.astype(jnp.int32)
    x0i = x0.astype(jnp.int32)
    n = boxes.shape[0]
    idx = (y0i[:, :, None] * W + x0i[:, None, :]).reshape(n, _PIX)
    ay = jnp.broadcast_to((1.0 - wy)[:, :, None], (n, _ROI, _ROI)).reshape(n, _PIX)
    by = jnp.broadcast_to(wy[:, :, None], (n, _ROI, _ROI)).reshape(n, _PIX)
    ax = jnp.broadcast_to((1.0 - wx)[:, None, :], (n, _ROI, _ROI)).reshape(n, _PIX)
    bx = jnp.broadcast_to(wx[:, None, :], (n, _ROI, _ROI)).reshape(n, _PIX)
    return idx, jnp.stack([ay, by, ax, bx], axis=0)


def _quad_table(fmap, W):
    t = fmap[0].reshape(-1, _C)
    hw = t.shape[0]
    tp = jnp.pad(t, ((0, W + 1), (0, 0)))
    return jnp.concatenate(
        [tp[:hw], tp[1:hw + 1], tp[W:hw + W], tp[W + 1:hw + W + 1]], axis=1)


def _pad_pt(a, n):
    return jnp.pad(a, ((0, _NP - n), (0, 0))).T


def _pad_t(a, n):
    k = a.shape[1]
    return jnp.pad(a.T, ((0, 0), (0, _NP - n))).reshape(k, _ROWS, 128)


def _ground_y(gp, x, z):
    return -(gp[0] * x + gp[2] * z + gp[3]) / gp[1]


def kernel(img_feat_map, bev_feat_map, top_anchors, image_shape,
           calibration_dict, ground_plane, img_mask, bev_mask,
           W_cls, b_cls, W_off, b_off, W_ang, b_ang):
    n = top_anchors.shape[0]
    gp = ground_plane

    bev_ins = _bev_norm_boxes(top_anchors)
    rgb_ins = _img_norm_boxes(top_anchors, image_shape, calibration_dict[0])
    boxes_img = jnp.take(rgb_ins, jnp.array([1, 0, 3, 2]), axis=1)
    boxes_bev = jnp.take(bev_ins, jnp.array([1, 0, 3, 2]), axis=1)

    hi, wi_ = img_feat_map.shape[1], img_feat_map.shape[2]
    hb, wb_ = bev_feat_map.shape[1], bev_feat_map.shape[2]
    idx_i, fac_i = _sample_coeffs(boxes_img, hi, wi_)
    idx_b, fac_b = _sample_coeffs(boxes_bev, hb, wb_)

    timg = _quad_table(img_mask * img_feat_map, wi_)
    tbev = _quad_table(bev_mask * bev_feat_map, wb_)

    ii = _pad_pt(idx_i, n)
    ib = _pad_pt(idx_b, n)
    wi4 = jnp.pad(fac_i, ((0, 0), (0, _NP - n), (0, 0))).transpose(0, 2, 1)
    wb4 = jnp.pad(fac_b, ((0, 0), (0, _NP - n), (0, 0))).transpose(0, 2, 1)

    f = _roi_fuse(timg, tbev, ii, ib, wi4, wb4)
    f2d = f.reshape(_C * _PIX, _NP)

    w_all = jnp.concatenate([W_cls, W_off, W_ang], axis=1)
    b_all = jnp.concatenate([b_cls, b_off, b_ang], axis=0)
    w_r = w_all.reshape(_PIX, _C, 16).transpose(1, 0, 2).reshape(_C * _PIX, 16)
    den = (img_mask + bev_mask).reshape(1)
    st = _mlp_t(f2d, w_r.T, b_all, den)

    anc_p = _pad_t(top_anchors, n)
    acc = _nms(anc_p, st.reshape(16, _ROWS, 128))
    top_idx = acc.reshape(-1)[:_NMS_OUT]

    s16 = st[:, :n].T
    obj = s16[:, 0:4]
    off = s16[:, 4:14]
    ang = s16[:, 14:16]
    obj_i = jnp.take(obj, top_idx, axis=0)
    top_scores_soft = jax.nn.softmax(obj_i, axis=1)
    ang_i = jnp.take(ang, top_idx, axis=0)
    top_orient = jnp.arctan2(ang_i[:, 1], ang_i[:, 0])

    a_i = jnp.take(top_anchors, top_idx, axis=0)
    o_i = jnp.take(off, top_idx, axis=0)
    x, y, z = a_i[:, 0], a_i[:, 1], a_i[:, 2]
    dx, dy, dz = a_i[:, 3], a_i[:, 4], a_i[:, 5]
    xs = jnp.stack([x - dx / 2, x + dx / 2, x + dx / 2, x - dx / 2], axis=1)
    zs = jnp.stack([z - dz / 2, z - dz / 2, z + dz / 2, z + dz / 2], axis=1)
    yg = _ground_y(gp, x, z)
    h1 = (y - dy / 2) - yg
    h2 = (y + dy / 2) - yg
    prop4cp = jnp.concatenate([xs, zs, h1[:, None], h2[:, None]], axis=1)
    pred4c = prop4cp + o_i
    pxs, pzs = pred4c[:, :4], pred4c[:, 4:8]
    ph1, ph2 = pred4c[:, 8], pred4c[:, 9]
    px, pz = pxs.mean(1), pzs.mean(1)
    pdx = pxs.max(1) - pxs.min(1)
    pdz = pzs.max(1) - pzs.min(1)
    pyg = _ground_y(gp, px, pz)
    py = pyg + (ph1 + ph2) / 2
    pdy = ph2 - ph1
    pred_anchors = jnp.stack([px, py, pz, pdx, pdy, pdz], axis=1)
    pred_box = jnp.concatenate(
        [pred_anchors, jnp.zeros((pred_anchors.shape[0], 1))], axis=1)

    return (top_scores_soft, (pred_anchors, pred4c, pred_box),
            top_orient, None)


# speedup vs baseline: 16.5473x; 1.2642x over previous
import functools

import jax
import jax.numpy as jnp
from jax import lax
from jax.experimental import pallas as pl
from jax.experimental.pallas import tpu as pltpu
from jax.experimental.pallas import tpu_sc as plsc

_EXT_X0, _EXT_X1 = -40.0, 40.0
_EXT_Z0, _EXT_Z1 = 0.0, 70.0
_ROI = 7
_PIX = _ROI * _ROI
_C = 32
_NMS_THR = 0.01
_NMS_OUT = 100
_N = 20000
_NP = 20480
_ROWS = _NP // 128
_NW = 32
_NT = _NP // _NW
_CH = 128
_NCHUNK = _NT // _CH

_SX = (-1.0, 1.0, 1.0, -1.0)
_SZ = (-1.0, -1.0, 1.0, 1.0)


def _roi_body(tboth, iboth, wi, wb, f_out,
              rows_v, idx_v, wi_v, wb_v, out_v, gsem, osem):
    wid = lax.axis_index("s") * 2 + lax.axis_index("c")
    tbase = wid * _NT
    cgbase = wid * _NCHUNK
    iota = lax.iota(jnp.int32, 16)
    n_t = _PIX * _NCHUNK

    def stage_and_fire(t, slot):
        p = t // _NCHUNK
        c = t % _NCHUNK
        pltpu.sync_copy(iboth.at[p, cgbase + c], idx_v.at[slot])
        pltpu.async_copy(tboth.at[idx_v.at[slot, 0]],
                         rows_v.at[slot, pl.ds(0, _CH)], gsem)
        pltpu.async_copy(tboth.at[idx_v.at[slot, 1]],
                         rows_v.at[slot, pl.ds(_CH, _CH)], gsem)

    stage_and_fire(0, 0)

    def t_loop(t, _):
        p = t // _NCHUNK
        c = t % _NCHUNK
        b = lax.rem(t, 2)
        po = lax.rem(p, 2)

        @pl.when(c == 0)
        def _():
            pltpu.sync_copy(wi.at[:, p, pl.ds(tbase, _NT)], wi_v)
            pltpu.sync_copy(wb.at[:, p, pl.ds(tbase, _NT)], wb_v)

            @pl.when(p >= 2)
            def _():
                pltpu.make_async_copy(
                    out_v.at[po],
                    f_out.at[:, pl.ds(p - 2, 1), pl.ds(tbase, _NT)], osem).wait()

        pltpu.make_async_copy(
            tboth.at[idx_v.at[b, 0]], rows_v.at[b, pl.ds(0, _CH)], gsem).wait()
        pltpu.make_async_copy(
            tboth.at[idx_v.at[b, 1]], rows_v.at[b, pl.ds(_CH, _CH)], gsem).wait()

        @pl.when(t + 1 < n_t)
        def _():
            stage_and_fire(t + 1, 1 - b)

        rows_b = rows_v.at[b]

        def g_loop(g, _):
            base = c * _CH + g * 16
            pix = g * 16 + iota
            pixb = pix + _CH
            ayi = wi_v[0, pl.ds(base, 16)]
            byi = wi_v[1, pl.ds(base, 16)]
            axi = wi_v[2, pl.ds(base, 16)]
            bxi = wi_v[3, pl.ds(base, 16)]
            ayb = wb_v[0, pl.ds(base, 16)]
            byb = wb_v[1, pl.ds(base, 16)]
            axb = wb_v[2, pl.ds(base, 16)]
            bxb = wb_v[3, pl.ds(base, 16)]
            for ch in range(_C):
                gi = [plsc.load_gather(
                    rows_b, [pix, jnp.full((16,), k * _C + ch, jnp.int32)])
                    for k in range(4)]
                gb = [plsc.load_gather(
                    rows_b, [pixb, jnp.full((16,), k * _C + ch, jnp.int32)])
                    for k in range(4)]
                a = (((gi[0] * ayi) * axi + (gi[1] * ayi) * bxi)
                     + (gi[2] * byi) * axi) + (gi[3] * byi) * bxi
                bb = (((gb[0] * ayb) * axb + (gb[1] * ayb) * bxb)
                      + (gb[2] * byb) * axb) + (gb[3] * byb) * bxb
                out_v[po, ch, 0, pl.ds(base, 16)] = a + bb
            return 0

        lax.fori_loop(0, _CH // 16, g_loop, 0)

        @pl.when(c == _NCHUNK - 1)
        def _():
            pltpu.async_copy(
                out_v.at[po], f_out.at[:, pl.ds(p, 1), pl.ds(tbase, _NT)], osem)
        return 0

    lax.fori_loop(0, n_t, t_loop, 0)

    for pp in (_PIX - 2, _PIX - 1):
        pltpu.make_async_copy(
            out_v.at[pp % 2], f_out.at[:, pl.ds(pp, 1), pl.ds(tbase, _NT)], osem).wait()


def _roi_fuse(tboth, iboth, wi, wb):
    mesh = plsc.VectorSubcoreMesh(core_axis_name="c", subcore_axis_name="s")
    k = functools.partial(
        pl.kernel,
        out_type=jax.ShapeDtypeStruct((_C, _PIX, _NP), jnp.float32),
        mesh=mesh,
        compiler_params=pltpu.CompilerParams(needs_layout_passes=False),
        scratch_types=[
            pltpu.VMEM((2, 2 * _CH, 128), jnp.float32),
            pltpu.VMEM((2, 2, _CH), jnp.int32),
            pltpu.VMEM((4, _NT), jnp.float32),
            pltpu.VMEM((4, _NT), jnp.float32),
            pltpu.VMEM((2, _C, 1, _NT), jnp.float32),
            pltpu.SemaphoreType.DMA,
            pltpu.SemaphoreType.DMA,
        ],
    )(_roi_body)
    return k(tboth, iboth, wi, wb)


def _mlp_body(den_ref, wt_ref, b_ref, x_ref, o_ref):
    xd = (x_ref[...] / den_ref[0]).astype(jnp.bfloat16)
    o_ref[...] = (
        jnp.dot(wt_ref[...].astype(jnp.bfloat16), xd,
                preferred_element_type=jnp.float32)
        + b_ref[...]
    )


def _mlp_t(f2d, wt, b, den, block=512):
    grid = _NP // block
    return pl.pallas_call(
        _mlp_body,
        grid=(grid,),
        in_specs=[
            pl.BlockSpec(memory_space=pltpu.SMEM),
            pl.BlockSpec((16, f2d.shape[0]), lambda i: (0, 0)),
            pl.BlockSpec((16, 1), lambda i: (0, 0)),
            pl.BlockSpec((f2d.shape[0], block), lambda i: (0, i)),
        ],
        out_specs=pl.BlockSpec((16, block), lambda i: (0, i)),
        out_shape=jax.ShapeDtypeStruct((16, _NP), jnp.float32),
    )(den, wt, b.reshape(16, 1), f2d)


def _nms_body(anc_ref, st_ref, out_ref):
    x = anc_ref[0]
    z = anc_ref[2]
    dx = anc_ref[3]
    dz = anc_ref[5]

    pxs = [x + 0.5 * _SX[k] * dx + st_ref[4 + k] for k in range(4)]
    pzs = [z + 0.5 * _SZ[k] * dz + st_ref[8 + k] for k in range(4)]
    xm = (pxs[0] + pxs[1] + pxs[2] + pxs[3]) * 0.25
    zm = (pzs[0] + pzs[1] + pzs[2] + pzs[3]) * 0.25
    dxn = jnp.maximum(jnp.maximum(pxs[0], pxs[1]), jnp.maximum(pxs[2], pxs[3])) - \
        jnp.minimum(jnp.minimum(pxs[0], pxs[1]), jnp.minimum(pxs[2], pxs[3]))
    dzn = jnp.maximum(jnp.maximum(pzs[0], pzs[1]), jnp.maximum(pzs[2], pzs[3])) - \
        jnp.minimum(jnp.minimum(pzs[0], pzs[1]), jnp.minimum(pzs[2], pzs[3]))
    bx1 = xm - dxn * 0.5
    bx2 = xm + dxn * 0.5
    bz1 = zm - dzn * 0.5
    bz2 = zm + dzn * 0.5
    areas = (bx2 - bx1) * (bz2 - bz1)

    scores = jnp.maximum(jnp.maximum(st_ref[1], st_ref[2]), st_ref[3])
    flatpos = (lax.broadcasted_iota(jnp.int32, (_ROWS, 128), 0) * 128
               + lax.broadcasted_iota(jnp.int32, (_ROWS, 128), 1))
    accpos = (lax.broadcasted_iota(jnp.int32, (8, 128), 0) * 128
              + lax.broadcasted_iota(jnp.int32, (8, 128), 1))

    def body(k, carry):
        valid, acc = carry
        s = jnp.where(valid > 0.5, scores, -1e30)
        m = jnp.max(s)
        idx = jnp.min(jnp.where(s == m, flatpos, jnp.int32(2147483647)))
        sel = flatpos == idx
        x1i = jnp.sum(jnp.where(sel, bx1, 0.0))
        z1i = jnp.sum(jnp.where(sel, bz1, 0.0))
        x2i = jnp.sum(jnp.where(sel, bx2, 0.0))
        z2i = jnp.sum(jnp.where(sel, bz2, 0.0))
        ai = jnp.sum(jnp.where(sel, areas, 0.0))
        xx1 = jnp.maximum(x1i, bx1)
        zz1 = jnp.maximum(z1i, bz1)
        xx2 = jnp.minimum(x2i, bx2)
        zz2 = jnp.minimum(z2i, bz2)
        inter = jnp.maximum(xx2 - xx1, 0.0) * jnp.maximum(zz2 - zz1, 0.0)
        iou = inter / (ai + areas - inter + 1e-8)
        keep = (iou <= _NMS_THR) & (~sel)
        valid = jnp.where(keep, valid, 0.0)
        acc = jnp.where(accpos == k, idx, acc)
        return valid, acc

    valid0 = (flatpos < _N).astype(jnp.float32)
    acc0 = jnp.zeros((8, 128), dtype=jnp.int32)
    _, acc = lax.fori_loop(0, _NMS_OUT, body, (valid0, acc0))
    out_ref[...] = acc


def _nms(anc, st):
    return pl.pallas_call(
        _nms_body,
        in_specs=[
            pl.BlockSpec((6, _ROWS, 128), lambda: (0, 0, 0)),
            pl.BlockSpec((16, _ROWS, 128), lambda: (0, 0, 0)),
        ],
        out_specs=pl.BlockSpec((8, 128), lambda: (0, 0)),
        out_shape=jax.ShapeDtypeStruct((8, 128), jnp.int32),
    )(anc, st)


def _bev_norm_boxes(anchors):
    x, z = anchors[:, 0], anchors[:, 2]
    dx, dz = anchors[:, 3], anchors[:, 5]
    u1 = (x - dx / 2 - _EXT_X0) / (_EXT_X1 - _EXT_X0)
    u2 = (x + dx / 2 - _EXT_X0) / (_EXT_X1 - _EXT_X0)
    v1 = (_EXT_Z1 - (z + dz / 2)) / (_EXT_Z1 - _EXT_Z0)
    v2 = (_EXT_Z1 - (z - dz / 2)) / (_EXT_Z1 - _EXT_Z0)
    return jnp.clip(jnp.stack([u1, v1, u2, v2], axis=1), 0.0, 1.0)


def _img_norm_boxes(anchors, image_shape, P):
    x, y, z = anchors[:, 0], anchors[:, 1], anchors[:, 2]
    dx, dy, dz = anchors[:, 3], anchors[:, 4], anchors[:, 5]
    offs = jnp.array([[sx, sy, sz] for sx in (-0.5, 0.5) for sy in (-0.5, 0.5)
                      for sz in (-0.5, 0.5)], dtype=jnp.float32)
    cx = x[:, None] + offs[None, :, 0] * dx[:, None]
    cy = y[:, None] + offs[None, :, 1] * dy[:, None]
    cz = z[:, None] + offs[None, :, 2] * dz[:, None]
    pts = jnp.stack([cx, cy, cz, jnp.ones_like(cx)], axis=-1)
    proj = jnp.einsum('nkj,ij->nki', pts, P)
    u = proj[..., 0] / jnp.maximum(proj[..., 2], 1e-3)
    v = proj[..., 1] / jnp.maximum(proj[..., 2], 1e-3)
    b = jnp.stack([u.min(1), v.min(1), u.max(1), v.max(1)], axis=1)
    H, W = image_shape[0], image_shape[1]
    b = b / jnp.stack([W, H, W, H])
    return jnp.clip(b, 0.0, 1.0)


def _sample_coeffs(boxes, H, W):
    y1, x1, y2, x2 = boxes[:, 0], boxes[:, 1], boxes[:, 2], boxes[:, 3]
    t = jnp.linspace(0.0, 1.0, _ROI)
    ys = (y1[:, None] + t[None, :] * (y2 - y1)[:, None]) * (H - 1)
    xs = (x1[:, None] + t[None, :] * (x2 - x1)[:, None]) * (W - 1)
    y0 = jnp.clip(jnp.floor(ys), 0, H - 2)
    x0 = jnp.clip(jnp.floor(xs), 0, W - 2)
    wy = jnp.clip(ys - y0, 0.0, 1.0)
    wx = jnp.clip(xs - x0, 0.0, 1.0)
    y0i = y0.astype(jnp.int32)
    x0i = x0.astype(jnp.int32)
    n = boxes.shape[0]
    idx = (y0i[:, :, None] * W + x0i[:, None, :]).reshape(n, _PIX)
    ay = jnp.broadcast_to((1.0 - wy)[:, :, None], (n, _ROI, _ROI)).reshape(n, _PIX)
    by = jnp.broadcast_to(wy[:, :, None], (n, _ROI, _ROI)).reshape(n, _PIX)
    ax = jnp.broadcast_to((1.0 - wx)[:, None, :], (n, _ROI, _ROI)).reshape(n, _PIX)
    bx = jnp.broadcast_to(wx[:, None, :], (n, _ROI, _ROI)).reshape(n, _PIX)
    return idx, jnp.stack([ay, by, ax, bx], axis=0)


def _quad_table(fmap, W):
    t = fmap[0].reshape(-1, _C)
    hw = t.shape[0]
    tp = jnp.pad(t, ((0, W + 1), (0, 0)))
    return jnp.concatenate(
        [tp[:hw], tp[1:hw + 1], tp[W:hw + W], tp[W + 1:hw + W + 1]], axis=1)


def _pad_pt(a, n):
    return jnp.pad(a, ((0, _NP - n), (0, 0))).T


def _pad_t(a, n):
    k = a.shape[1]
    return jnp.pad(a.T, ((0, 0), (0, _NP - n))).reshape(k, _ROWS, 128)


def _ground_y(gp, x, z):
    return -(gp[0] * x + gp[2] * z + gp[3]) / gp[1]


def kernel(img_feat_map, bev_feat_map, top_anchors, image_shape,
           calibration_dict, ground_plane, img_mask, bev_mask,
           W_cls, b_cls, W_off, b_off, W_ang, b_ang):
    n = top_anchors.shape[0]
    gp = ground_plane

    bev_ins = _bev_norm_boxes(top_anchors)
    rgb_ins = _img_norm_boxes(top_anchors, image_shape, calibration_dict[0])
    boxes_img = jnp.take(rgb_ins, jnp.array([1, 0, 3, 2]), axis=1)
    boxes_bev = jnp.take(bev_ins, jnp.array([1, 0, 3, 2]), axis=1)

    hi, wi_ = img_feat_map.shape[1], img_feat_map.shape[2]
    hb, wb_ = bev_feat_map.shape[1], bev_feat_map.shape[2]
    idx_i, fac_i = _sample_coeffs(boxes_img, hi, wi_)
    idx_b, fac_b = _sample_coeffs(boxes_bev, hb, wb_)

    timg = _quad_table(img_mask * img_feat_map, wi_)
    tbev = _quad_table(bev_mask * bev_feat_map, wb_)
    tboth = jnp.concatenate([timg, tbev], axis=0)

    ii = _pad_pt(idx_i, n)
    ib = _pad_pt(idx_b, n) + timg.shape[0]
    iboth = jnp.stack(
        [ii.reshape(_PIX, _NP // _CH, _CH), ib.reshape(_PIX, _NP // _CH, _CH)],
        axis=2)
    wi4 = jnp.pad(fac_i, ((0, 0), (0, _NP - n), (0, 0))).transpose(0, 2, 1)
    wb4 = jnp.pad(fac_b, ((0, 0), (0, _NP - n), (0, 0))).transpose(0, 2, 1)

    f = _roi_fuse(tboth, iboth, wi4, wb4)
    f2d = f.reshape(_C * _PIX, _NP)

    w_all = jnp.concatenate([W_cls, W_off, W_ang], axis=1)
    b_all = jnp.concatenate([b_cls, b_off, b_ang], axis=0)
    w_r = w_all.reshape(_PIX, _C, 16).transpose(1, 0, 2).reshape(_C * _PIX, 16)
    den = (img_mask + bev_mask).reshape(1)
    st = _mlp_t(f2d, w_r.T, b_all, den)

    anc_p = _pad_t(top_anchors, n)
    acc = _nms(anc_p, st.reshape(16, _ROWS, 128))
    top_idx = acc.reshape(-1)[:_NMS_OUT]

    s16 = st[:, :n].T
    obj = s16[:, 0:4]
    off = s16[:, 4:14]
    ang = s16[:, 14:16]
    obj_i = jnp.take(obj, top_idx, axis=0)
    top_scores_soft = jax.nn.softmax(obj_i, axis=1)
    ang_i = jnp.take(ang, top_idx, axis=0)
    top_orient = jnp.arctan2(ang_i[:, 1], ang_i[:, 0])

    a_i = jnp.take(top_anchors, top_idx, axis=0)
    o_i = jnp.take(off, top_idx, axis=0)
    x, y, z = a_i[:, 0], a_i[:, 1], a_i[:, 2]
    dx, dy, dz = a_i[:, 3], a_i[:, 4], a_i[:, 5]
    xs = jnp.stack([x - dx / 2, x + dx / 2, x + dx / 2, x - dx / 2], axis=1)
    zs = jnp.stack([z - dz / 2, z - dz / 2, z + dz / 2, z + dz / 2], axis=1)
    yg = _ground_y(gp, x, z)
    h1 = (y - dy / 2) - yg
    h2 = (y + dy / 2) - yg
    prop4cp = jnp.concatenate([xs, zs, h1[:, None], h2[:, None]], axis=1)
    pred4c = prop4cp + o_i
    pxs, pzs = pred4c[:, :4], pred4c[:, 4:8]
    ph1, ph2 = pred4c[:, 8], pred4c[:, 9]
    px, pz = pxs.mean(1), pzs.mean(1)
    pdx = pxs.max(1) - pxs.min(1)
    pdz = pzs.max(1) - pzs.min(1)
    pyg = _ground_y(gp, px, pz)
    py = pyg + (ph1 + ph2) / 2
    pdy = ph2 - ph1
    pred_anchors = jnp.stack([px, py, pz, pdx, pdy, pdz], axis=1)
    pred_box = jnp.concatenate(
        [pred_anchors, jnp.zeros((pred_anchors.shape[0], 1))], axis=1)

    return (top_scores_soft, (pred_anchors, pred4c, pred_box),
            top_orient, None)


# speedup vs baseline: 31.7558x; 1.9191x over previous
import functools

import jax
import jax.numpy as jnp
from jax import lax
from jax.experimental import pallas as pl
from jax.experimental.pallas import tpu as pltpu
from jax.experimental.pallas import tpu_sc as plsc

_EXT_X0, _EXT_X1 = -40.0, 40.0
_EXT_Z0, _EXT_Z1 = 0.0, 70.0
_ROI = 7
_PIX = _ROI * _ROI
_C = 32
_NMS_THR = 0.01
_NMS_OUT = 100
_N = 20000
_NP = 20480
_ROWS = _NP // 128
_NW = 32
_NT = _NP // _NW
_CH = 128
_NCHUNK = _NT // _CH

_SX = (-1.0, 1.0, 1.0, -1.0)
_SZ = (-1.0, -1.0, 1.0, 1.0)


def _roi_body(tboth, iboth, wi, wb, f_out,
              rows_v, idx_v, wi_v, wb_v, out_v, gsem, osem):
    wid = lax.axis_index("s") * 2 + lax.axis_index("c")
    tbase = wid * _NT
    cgbase = wid * _NCHUNK
    iota = lax.iota(jnp.int32, 16)
    n_t = _PIX * _NCHUNK

    def stage_and_fire(t, slot):
        p = t // _NCHUNK
        c = t % _NCHUNK
        pltpu.sync_copy(iboth.at[p, cgbase + c], idx_v.at[slot])
        pltpu.async_copy(tboth.at[idx_v.at[slot, 0]],
                         rows_v.at[slot, pl.ds(0, _CH)], gsem)
        pltpu.async_copy(tboth.at[idx_v.at[slot, 1]],
                         rows_v.at[slot, pl.ds(_CH, _CH)], gsem)

    stage_and_fire(0, 0)

    def t_loop(t, _):
        p = t // _NCHUNK
        c = t % _NCHUNK
        b = lax.rem(t, 2)
        po = lax.rem(p, 2)

        @pl.when(c == 0)
        def _():
            pltpu.sync_copy(wi.at[:, p, pl.ds(tbase, _NT)], wi_v)
            pltpu.sync_copy(wb.at[:, p, pl.ds(tbase, _NT)], wb_v)

            @pl.when(p >= 2)
            def _():
                pltpu.make_async_copy(
                    out_v.at[po],
                    f_out.at[pl.ds(p - 2, 1), pl.ds(tbase * _C, _NT * _C)],
                    osem).wait()

        pltpu.make_async_copy(
            tboth.at[idx_v.at[b, 0]], rows_v.at[b, pl.ds(0, _CH)], gsem).wait()
        pltpu.make_async_copy(
            tboth.at[idx_v.at[b, 1]], rows_v.at[b, pl.ds(_CH, _CH)], gsem).wait()

        @pl.when(t + 1 < n_t)
        def _():
            stage_and_fire(t + 1, 1 - b)

        rows_b = rows_v.at[b]
        z16 = jnp.zeros((16,), jnp.int32)

        def g_loop(g, _):
            for j in range(16):
                jr = g * 16 + j
                nl = c * _CH + jr
                col = z16 + nl
                wbc = [plsc.load_gather(wi_v, [z16 + r, col]) for r in range(4)]
                wbb = [plsc.load_gather(wb_v, [z16 + r, col]) for r in range(4)]
                ayi, byi, axi, bxi = wbc
                ayb, byb, axb, bxb = wbb
                for h in range(2):
                    vi = [rows_b[jr, pl.ds(k * _C + h * 16, 16)]
                          for k in range(4)]
                    vb = [rows_b[jr + _CH, pl.ds(k * _C + h * 16, 16)]
                          for k in range(4)]
                    a = (((vi[0] * ayi) * axi + (vi[1] * ayi) * bxi)
                         + (vi[2] * byi) * axi) + (vi[3] * byi) * bxi
                    bb = (((vb[0] * ayb) * axb + (vb[1] * ayb) * bxb)
                          + (vb[2] * byb) * axb) + (vb[3] * byb) * bxb
                    out_v[po, 0, pl.ds(nl * _C + h * 16, 16)] = a + bb
            return 0

        lax.fori_loop(0, _CH // 16, g_loop, 0)

        @pl.when(c == _NCHUNK - 1)
        def _():
            pltpu.async_copy(
                out_v.at[po],
                f_out.at[pl.ds(p, 1), pl.ds(tbase * _C, _NT * _C)], osem)
        return 0

    lax.fori_loop(0, n_t, t_loop, 0)

    for pp in (_PIX - 2, _PIX - 1):
        pltpu.make_async_copy(
            out_v.at[pp % 2],
            f_out.at[pl.ds(pp, 1), pl.ds(tbase * _C, _NT * _C)], osem).wait()


def _roi_fuse(tboth, iboth, wi, wb):
    mesh = plsc.VectorSubcoreMesh(core_axis_name="c", subcore_axis_name="s")
    k = functools.partial(
        pl.kernel,
        out_type=jax.ShapeDtypeStruct((_PIX, _NP * _C), jnp.float32),
        mesh=mesh,
        compiler_params=pltpu.CompilerParams(needs_layout_passes=False),
        scratch_types=[
            pltpu.VMEM((2, 2 * _CH, 128), jnp.float32),
            pltpu.VMEM((2, 2, _CH), jnp.int32),
            pltpu.VMEM((4, _NT), jnp.float32),
            pltpu.VMEM((4, _NT), jnp.float32),
            pltpu.VMEM((2, 1, _NT * _C), jnp.float32),
            pltpu.SemaphoreType.DMA,
            pltpu.SemaphoreType.DMA,
        ],
    )(_roi_body)
    return k(tboth, iboth, wi, wb)


def _mlp_body(den_ref, wt_ref, b_ref, x_ref, o_ref):
    xd = (x_ref[...] / den_ref[0]).astype(jnp.bfloat16)
    o_ref[...] = (
        jnp.dot(wt_ref[...].astype(jnp.bfloat16), xd,
                preferred_element_type=jnp.float32)
        + b_ref[...]
    )


def _mlp_t(f2d, wt, b, den, block=512):
    grid = _NP // block
    return pl.pallas_call(
        _mlp_body,
        grid=(grid,),
        in_specs=[
            pl.BlockSpec(memory_space=pltpu.SMEM),
            pl.BlockSpec((16, f2d.shape[0]), lambda i: (0, 0)),
            pl.BlockSpec((16, 1), lambda i: (0, 0)),
            pl.BlockSpec((f2d.shape[0], block), lambda i: (0, i)),
        ],
        out_specs=pl.BlockSpec((16, block), lambda i: (0, i)),
        out_shape=jax.ShapeDtypeStruct((16, _NP), jnp.float32),
    )(den, wt, b.reshape(16, 1), f2d)


def _nms_body(anc_ref, st_ref, out_ref):
    x = anc_ref[0]
    z = anc_ref[2]
    dx = anc_ref[3]
    dz = anc_ref[5]

    pxs = [x + 0.5 * _SX[k] * dx + st_ref[4 + k] for k in range(4)]
    pzs = [z + 0.5 * _SZ[k] * dz + st_ref[8 + k] for k in range(4)]
    xm = (pxs[0] + pxs[1] + pxs[2] + pxs[3]) * 0.25
    zm = (pzs[0] + pzs[1] + pzs[2] + pzs[3]) * 0.25
    dxn = jnp.maximum(jnp.maximum(pxs[0], pxs[1]), jnp.maximum(pxs[2], pxs[3])) - \
        jnp.minimum(jnp.minimum(pxs[0], pxs[1]), jnp.minimum(pxs[2], pxs[3]))
    dzn = jnp.maximum(jnp.maximum(pzs[0], pzs[1]), jnp.maximum(pzs[2], pzs[3])) - \
        jnp.minimum(jnp.minimum(pzs[0], pzs[1]), jnp.minimum(pzs[2], pzs[3]))
    bx1 = xm - dxn * 0.5
    bx2 = xm + dxn * 0.5
    bz1 = zm - dzn * 0.5
    bz2 = zm + dzn * 0.5
    areas = (bx2 - bx1) * (bz2 - bz1)

    scores = jnp.maximum(jnp.maximum(st_ref[1], st_ref[2]), st_ref[3])
    flatpos = (lax.broadcasted_iota(jnp.int32, (_ROWS, 128), 0) * 128
               + lax.broadcasted_iota(jnp.int32, (_ROWS, 128), 1))
    accpos = (lax.broadcasted_iota(jnp.int32, (8, 128), 0) * 128
              + lax.broadcasted_iota(jnp.int32, (8, 128), 1))

    def body(k, carry):
        valid, acc = carry
        s = jnp.where(valid > 0.5, scores, -1e30)
        m = jnp.max(s)
        idx = jnp.min(jnp.where(s == m, flatpos, jnp.int32(2147483647)))
        sel = flatpos == idx
        x1i = jnp.sum(jnp.where(sel, bx1, 0.0))
        z1i = jnp.sum(jnp.where(sel, bz1, 0.0))
        x2i = jnp.sum(jnp.where(sel, bx2, 0.0))
        z2i = jnp.sum(jnp.where(sel, bz2, 0.0))
        ai = jnp.sum(jnp.where(sel, areas, 0.0))
        xx1 = jnp.maximum(x1i, bx1)
        zz1 = jnp.maximum(z1i, bz1)
        xx2 = jnp.minimum(x2i, bx2)
        zz2 = jnp.minimum(z2i, bz2)
        inter = jnp.maximum(xx2 - xx1, 0.0) * jnp.maximum(zz2 - zz1, 0.0)
        iou = inter / (ai + areas - inter + 1e-8)
        keep = (iou <= _NMS_THR) & (~sel)
        valid = jnp.where(keep, valid, 0.0)
        acc = jnp.where(accpos == k, idx, acc)
        return valid, acc

    valid0 = (flatpos < _N).astype(jnp.float32)
    acc0 = jnp.zeros((8, 128), dtype=jnp.int32)
    _, acc = lax.fori_loop(0, _NMS_OUT, body, (valid0, acc0))
    out_ref[...] = acc


def _nms(anc, st):
    return pl.pallas_call(
        _nms_body,
        in_specs=[
            pl.BlockSpec((6, _ROWS, 128), lambda: (0, 0, 0)),
            pl.BlockSpec((16, _ROWS, 128), lambda: (0, 0, 0)),
        ],
        out_specs=pl.BlockSpec((8, 128), lambda: (0, 0)),
        out_shape=jax.ShapeDtypeStruct((8, 128), jnp.int32),
    )(anc, st)


def _bev_norm_boxes(anchors):
    x, z = anchors[:, 0], anchors[:, 2]
    dx, dz = anchors[:, 3], anchors[:, 5]
    u1 = (x - dx / 2 - _EXT_X0) / (_EXT_X1 - _EXT_X0)
    u2 = (x + dx / 2 - _EXT_X0) / (_EXT_X1 - _EXT_X0)
    v1 = (_EXT_Z1 - (z + dz / 2)) / (_EXT_Z1 - _EXT_Z0)
    v2 = (_EXT_Z1 - (z - dz / 2)) / (_EXT_Z1 - _EXT_Z0)
    return jnp.clip(jnp.stack([u1, v1, u2, v2], axis=1), 0.0, 1.0)


def _img_norm_boxes(anchors, image_shape, P):
    x, y, z = anchors[:, 0], anchors[:, 1], anchors[:, 2]
    dx, dy, dz = anchors[:, 3], anchors[:, 4], anchors[:, 5]
    offs = jnp.array([[sx, sy, sz] for sx in (-0.5, 0.5) for sy in (-0.5, 0.5)
                      for sz in (-0.5, 0.5)], dtype=jnp.float32)
    cx = x[:, None] + offs[None, :, 0] * dx[:, None]
    cy = y[:, None] + offs[None, :, 1] * dy[:, None]
    cz = z[:, None] + offs[None, :, 2] * dz[:, None]
    pts = jnp.stack([cx, cy, cz, jnp.ones_like(cx)], axis=-1)
    proj = jnp.einsum('nkj,ij->nki', pts, P)
    u = proj[..., 0] / jnp.maximum(proj[..., 2], 1e-3)
    v = proj[..., 1] / jnp.maximum(proj[..., 2], 1e-3)
    b = jnp.stack([u.min(1), v.min(1), u.max(1), v.max(1)], axis=1)
    H, W = image_shape[0], image_shape[1]
    b = b / jnp.stack([W, H, W, H])
    return jnp.clip(b, 0.0, 1.0)


def _sample_coeffs(boxes, H, W):
    y1, x1, y2, x2 = boxes[:, 0], boxes[:, 1], boxes[:, 2], boxes[:, 3]
    t = jnp.linspace(0.0, 1.0, _ROI)
    ys = (y1[:, None] + t[None, :] * (y2 - y1)[:, None]) * (H - 1)
    xs = (x1[:, None] + t[None, :] * (x2 - x1)[:, None]) * (W - 1)
    y0 = jnp.clip(jnp.floor(ys), 0, H - 2)
    x0 = jnp.clip(jnp.floor(xs), 0, W - 2)
    wy = jnp.clip(ys - y0, 0.0, 1.0)
    wx = jnp.clip(xs - x0, 0.0, 1.0)
    y0i = y0.astype(jnp.int32)
    x0i = x0.astype(jnp.int32)
    n = boxes.shape[0]
    idx = (y0i[:, :, None] * W + x0i[:, None, :]).reshape(n, _PIX)
    ay = jnp.broadcast_to((1.0 - wy)[:, :, None], (n, _ROI, _ROI)).reshape(n, _PIX)
    by = jnp.broadcast_to(wy[:, :, None], (n, _ROI, _ROI)).reshape(n, _PIX)
    ax = jnp.broadcast_to((1.0 - wx)[:, None, :], (n, _ROI, _ROI)).reshape(n, _PIX)
    bx = jnp.broadcast_to(wx[:, None, :], (n, _ROI, _ROI)).reshape(n, _PIX)
    return idx, jnp.stack([ay, by, ax, bx], axis=0)


def _quad_table(fmap, W):
    t = fmap[0].reshape(-1, _C)
    hw = t.shape[0]
    tp = jnp.pad(t, ((0, W + 1), (0, 0)))
    return jnp.concatenate(
        [tp[:hw], tp[1:hw + 1], tp[W:hw + W], tp[W + 1:hw + W + 1]], axis=1)


def _pad_pt(a, n):
    return jnp.pad(a, ((0, _NP - n), (0, 0))).T


def _pad_t(a, n):
    k = a.shape[1]
    return jnp.pad(a.T, ((0, 0), (0, _NP - n))).reshape(k, _ROWS, 128)


def _ground_y(gp, x, z):
    return -(gp[0] * x + gp[2] * z + gp[3]) / gp[1]


def kernel(img_feat_map, bev_feat_map, top_anchors, image_shape,
           calibration_dict, ground_plane, img_mask, bev_mask,
           W_cls, b_cls, W_off, b_off, W_ang, b_ang):
    n = top_anchors.shape[0]
    gp = ground_plane

    bev_ins = _bev_norm_boxes(top_anchors)
    rgb_ins = _img_norm_boxes(top_anchors, image_shape, calibration_dict[0])
    boxes_img = jnp.take(rgb_ins, jnp.array([1, 0, 3, 2]), axis=1)
    boxes_bev = jnp.take(bev_ins, jnp.array([1, 0, 3, 2]), axis=1)

    hi, wi_ = img_feat_map.shape[1], img_feat_map.shape[2]
    hb, wb_ = bev_feat_map.shape[1], bev_feat_map.shape[2]
    idx_i, fac_i = _sample_coeffs(boxes_img, hi, wi_)
    idx_b, fac_b = _sample_coeffs(boxes_bev, hb, wb_)

    timg = _quad_table(img_mask * img_feat_map, wi_)
    tbev = _quad_table(bev_mask * bev_feat_map, wb_)
    tboth = jnp.concatenate([timg, tbev], axis=0)

    ii = _pad_pt(idx_i, n)
    ib = _pad_pt(idx_b, n) + timg.shape[0]
    iboth = jnp.stack(
        [ii.reshape(_PIX, _NP // _CH, _CH), ib.reshape(_PIX, _NP // _CH, _CH)],
        axis=2)
    wi4 = jnp.pad(fac_i, ((0, 0), (0, _NP - n), (0, 0))).transpose(0, 2, 1)
    wb4 = jnp.pad(fac_b, ((0, 0), (0, _NP - n), (0, 0))).transpose(0, 2, 1)

    f = _roi_fuse(tboth, iboth, wi4, wb4)
    f2d = f.reshape(_PIX, _NP, _C).transpose(0, 2, 1).reshape(_PIX * _C, _NP)

    w_all = jnp.concatenate([W_cls, W_off, W_ang], axis=1)
    b_all = jnp.concatenate([b_cls, b_off, b_ang], axis=0)
    den = (img_mask + bev_mask).reshape(1)
    st = _mlp_t(f2d, w_all.T, b_all, den)

    anc_p = _pad_t(top_anchors, n)
    acc = _nms(anc_p, st.reshape(16, _ROWS, 128))
    top_idx = acc.reshape(-1)[:_NMS_OUT]

    s16 = st[:, :n].T
    obj = s16[:, 0:4]
    off = s16[:, 4:14]
    ang = s16[:, 14:16]
    obj_i = jnp.take(obj, top_idx, axis=0)
    top_scores_soft = jax.nn.softmax(obj_i, axis=1)
    ang_i = jnp.take(ang, top_idx, axis=0)
    top_orient = jnp.arctan2(ang_i[:, 1], ang_i[:, 0])

    a_i = jnp.take(top_anchors, top_idx, axis=0)
    o_i = jnp.take(off, top_idx, axis=0)
    x, y, z = a_i[:, 0], a_i[:, 1], a_i[:, 2]
    dx, dy, dz = a_i[:, 3], a_i[:, 4], a_i[:, 5]
    xs = jnp.stack([x - dx / 2, x + dx / 2, x + dx / 2, x - dx / 2], axis=1)
    zs = jnp.stack([z - dz / 2, z - dz / 2, z + dz / 2, z + dz / 2], axis=1)
    yg = _ground_y(gp, x, z)
    h1 = (y - dy / 2) - yg
    h2 = (y + dy / 2) - yg
    prop4cp = jnp.concatenate([xs, zs, h1[:, None], h2[:, None]], axis=1)
    pred4c = prop4cp + o_i
    pxs, pzs = pred4c[:, :4], pred4c[:, 4:8]
    ph1, ph2 = pred4c[:, 8], pred4c[:, 9]
    px, pz = pxs.mean(1), pzs.mean(1)
    pdx = pxs.max(1) - pxs.min(1)
    pdz = pzs.max(1) - pzs.min(1)
    pyg = _ground_y(gp, px, pz)
    py = pyg + (ph1 + ph2) / 2
    pdy = ph2 - ph1
    pred_anchors = jnp.stack([px, py, pz, pdx, pdy, pdz], axis=1)
    pred_box = jnp.concatenate(
        [pred_anchors, jnp.zeros((pred_anchors.shape[0], 1))], axis=1)

    return (top_scores_soft, (pred_anchors, pred4c, pred_box),
            top_orient, None)
